# Initial kernel scaffold; baseline (speedup 1.0000x reference)
import jax
import jax.numpy as jnp
from jax.experimental import pallas as pl


def kernel(x, queue):
    raise NotImplementedError("write your pallas kernel here")


# speedup vs baseline: 1.1084x; 1.1084x over previous
import functools

import jax
import jax.numpy as jnp
from jax import lax
from jax.experimental import pallas as pl
from jax.experimental.pallas import tpu as pltpu
from jax.experimental.pallas import tpu_sc as plsc

_EPS = 1e-12


def _norm_body(a_ref, o_ref):
    t = a_ref[...]
    n = jnp.sqrt(jnp.sum(t * t, axis=1, keepdims=True))
    o_ref[...] = (t / jnp.maximum(n, _EPS)).astype(o_ref.dtype)


def _normalize_rows(a, row_tile, out_dtype=None):
    m, d = a.shape
    assert m % row_tile == 0
    return pl.pallas_call(
        _norm_body,
        grid=(m // row_tile,),
        in_specs=[pl.BlockSpec((row_tile, d), lambda i: (i, 0))],
        out_specs=pl.BlockSpec((row_tile, d), lambda i: (i, 0)),
        out_shape=jax.ShapeDtypeStruct((m, d), out_dtype or a.dtype),
    )(a)


def _argmax_body(x_ref, q_ref, idx_ref, bestv_ref, besti_ref, *, kt, nk):
    k = pl.program_id(1)
    q_t = q_ref[...]
    n = jnp.sqrt(jnp.sum(q_t * q_t, axis=1, keepdims=True))
    qn = (q_t / jnp.maximum(n, _EPS)).astype(jnp.bfloat16)
    sim = lax.dot_general(
        x_ref[...], qn, (((1,), (1,)), ((), ())),
        preferred_element_type=jnp.float32,
    )
    m = jnp.max(sim, axis=1, keepdims=True)
    ii = lax.broadcasted_iota(jnp.int32, sim.shape, 1)
    loc = jnp.min(jnp.where(sim == m, ii, jnp.int32(2**30)), axis=1)
    loc = loc + k * kt
    val = m[:, 0]

    @pl.when(k == 0)
    def _():
        bestv_ref[...] = val
        besti_ref[...] = loc

    @pl.when(k > 0)
    def _():
        upd = val > bestv_ref[...]
        bestv_ref[...] = jnp.where(upd, val, bestv_ref[...])
        besti_ref[...] = jnp.where(upd, loc, besti_ref[...])

    @pl.when(k == nk - 1)
    def _():
        idx_ref[0, 0, :] = besti_ref[...]


def _nn_argmax(x_l2, queue, qt, kt):
    q, d = x_l2.shape
    k_total = queue.shape[0]
    assert q % qt == 0 and k_total % kt == 0
    nq, nk = q // qt, k_total // kt
    idx3 = pl.pallas_call(
        functools.partial(_argmax_body, kt=kt, nk=nk),
        grid=(nq, nk),
        in_specs=[
            pl.BlockSpec((qt, d), lambda i, j: (i, 0)),
            pl.BlockSpec((kt, d), lambda i, j: (j, 0)),
        ],
        out_specs=pl.BlockSpec((1, 1, qt), lambda i, j: (i, 0, 0)),
        out_shape=jax.ShapeDtypeStruct((nq, 1, qt), jnp.int32),
        scratch_shapes=[
            pltpu.VMEM((qt,), jnp.float32),
            pltpu.VMEM((qt,), jnp.int32),
        ],
    )(x_l2, queue)
    return idx3.reshape(q)


def _sc_gather(table, idx):
    b = idx.shape[0]
    v, d = table.shape
    info = plsc.get_sparse_core_info()
    nw = info.num_cores * info.num_subcores
    assert b % (8 * nw) == 0 and d % info.num_lanes == 0
    b_per_w = b // nw
    mesh = plsc.VectorSubcoreMesh(core_axis_name="c", subcore_axis_name="s")

    @functools.partial(
        pl.kernel, mesh=mesh,
        out_type=jax.ShapeDtypeStruct((b, d), table.dtype),
        scratch_types=[
            pltpu.VMEM((b_per_w,), jnp.int32),
            pltpu.VMEM((b_per_w, d), table.dtype),
            pltpu.SemaphoreType.DMA,
        ],
    )
    def k(table_hbm, idx_hbm, out_hbm, idx_v, rows_v, sem):
        wid = lax.axis_index("s") * info.num_cores + lax.axis_index("c")
        base = wid * b_per_w
        pltpu.sync_copy(idx_hbm.at[pl.ds(base, b_per_w)], idx_v)
        pltpu.async_copy(table_hbm.at[idx_v], rows_v, sem).wait()
        pltpu.sync_copy(rows_v, out_hbm.at[pl.ds(base, b_per_w)])

    return k(table, idx)


def kernel(x, queue):
    q, d = x.shape
    x_l2 = _normalize_rows(x, row_tile=min(1024, q), out_dtype=jnp.bfloat16)
    nn_idx = _nn_argmax(x_l2, queue, qt=min(1024, q), kt=2000 if queue.shape[0] % 2000 == 0 else queue.shape[0])
    rows = _sc_gather(queue, nn_idx)
    return _normalize_rows(rows, row_tile=min(1024, q))
